# Initial kernel scaffold; baseline (speedup 1.0000x reference)
import functools

import jax
import jax.numpy as jnp
from jax import lax
from jax.experimental import pallas as pl
from jax.experimental.pallas import tpu as pltpu
from jax.experimental.pallas import tpu_sc as plsc

N = 10000
E = 320000
D_IN = 128

NC = 2
NS = 16
CHUNK = 128
NTILES = NC * NS

K = -(-E // (NTILES * CHUNK))
EPAD = NTILES * CHUNK * K
EPT = K * CHUNK

NP = 10240
ROWS_PT = NP // NS
NCOPY = ROWS_PT // CHUNK

BN = 1024
GRID = NP // BN

_MESH = dict(core_axis_name="c", subcore_axis_name="s")


def _make_agg(d):
    nlan = d // 16

    @functools.partial(
        pl.kernel,
        out_type=jax.ShapeDtypeStruct((NC, NP, d), jnp.float32),
        mesh=plsc.VectorSubcoreMesh(**_MESH),
        scratch_types=[
            pltpu.VMEM((CHUNK,), jnp.int32),
            pltpu.VMEM((CHUNK,), jnp.int32),
            pltpu.VMEM((CHUNK, d), jnp.float32),
            pltpu.VMEM_SHARED((NP, d), jnp.float32),
            pltpu.SemaphoreType.DMA,
        ],
        name=f"gcn_agg_d{d}",
    )
    def agg(src_hbm, dst_hbm, g_hbm, out_hbm, sidx, didx, rows, acc, sem):
        c = lax.axis_index("c")
        s = lax.axis_index("s")

        def zrow(i, _):
            rows[i // nlan, pl.ds((i % nlan) * 16, 16)] = jnp.zeros(
                (16,), jnp.float32)
            return 0
        lax.fori_loop(0, CHUNK * nlan, zrow, 0)
        row0 = s * ROWS_PT
        for j in range(NCOPY):
            pltpu.sync_copy(rows, acc.at[pl.ds(row0 + j * CHUNK, CHUNK)])
        plsc.subcore_barrier()

        base0 = (c * NS + s) * EPT

        def body(k, _):
            b = base0 + k * CHUNK
            pltpu.sync_copy(src_hbm.at[pl.ds(b, CHUNK)], sidx)
            pltpu.sync_copy(dst_hbm.at[pl.ds(b, CHUNK)], didx)
            pltpu.async_copy(g_hbm.at[sidx], rows, sem).wait()
            pltpu.sync_copy(rows, acc.at[didx], add=True)
            return 0
        lax.fori_loop(0, K, body, 0)

        plsc.subcore_barrier()
        pltpu.sync_copy(acc.at[pl.ds(row0, ROWS_PT)],
                        out_hbm.at[c, pl.ds(row0, ROWS_PT)])

    return agg


_agg64 = _make_agg(64)
_agg32 = _make_agg(32)
_agg16 = _make_agg(16)


@functools.partial(
    pl.kernel,
    out_type=jax.ShapeDtypeStruct((NC, NP, 16), jnp.float32),
    mesh=plsc.VectorSubcoreMesh(**_MESH),
    scratch_types=[
        pltpu.VMEM((CHUNK,), jnp.int32),
        pltpu.VMEM((CHUNK, 16), jnp.float32),
        pltpu.VMEM((CHUNK, 16), jnp.float32),
        pltpu.VMEM_SHARED((NP, 16), jnp.float32),
    ],
    name="gcn_degree",
)
def _deg_kernel(dst_hbm, out_hbm, didx, ones, zeros, acc):
    c = lax.axis_index("c")
    s = lax.axis_index("s")

    def fill(i, _):
        zeros[i, pl.ds(0, 16)] = jnp.zeros((16,), jnp.float32)
        ones[i, pl.ds(0, 16)] = jnp.ones((16,), jnp.float32)
        return 0
    lax.fori_loop(0, CHUNK, fill, 0)
    row0 = s * ROWS_PT
    for j in range(NCOPY):
        pltpu.sync_copy(zeros, acc.at[pl.ds(row0 + j * CHUNK, CHUNK)])
    plsc.subcore_barrier()

    base0 = (c * NS + s) * EPT

    def body(k, _):
        b = base0 + k * CHUNK
        pltpu.sync_copy(dst_hbm.at[pl.ds(b, CHUNK)], didx)
        pltpu.sync_copy(ones, acc.at[didx], add=True)
        return 0
    lax.fori_loop(0, K, body, 0)

    plsc.subcore_barrier()
    pltpu.sync_copy(acc.at[pl.ds(row0, ROWS_PT)],
                    out_hbm.at[c, pl.ds(row0, ROWS_PT)])


def _tc1_body(degp_ref, x_ref, w_ref, dinv_ref, g_ref):
    degp = degp_ref[...]
    deg = degp[0, :, 0:1] + degp[1, :, 0:1] + 1.0
    dinv = lax.rsqrt(deg)
    dinv_ref[...] = dinv
    g_ref[...] = dinv * jnp.dot(x_ref[...], w_ref[...],
                                preferred_element_type=jnp.float32)


def _tc_mid_body(sp_ref, g_ref, dinv_ref, b_ref, w_ref, gout_ref):
    dinv = dinv_ref[...]
    stot = sp_ref[0] + sp_ref[1] + g_ref[...]
    xn = jnp.maximum(dinv * stot + b_ref[...], 0.0)
    gout_ref[...] = dinv * jnp.dot(xn, w_ref[...],
                                   preferred_element_type=jnp.float32)


def _tc_last_body(sp_ref, g_ref, dinv_ref, b_ref, out_ref):
    out_ref[...] = dinv_ref[...] * (sp_ref[0] + sp_ref[1] + g_ref[...]) \
        + b_ref[...]


def _row_spec(d):
    return pl.BlockSpec((BN, d), lambda i: (i, 0))


def _part_spec(d):
    return pl.BlockSpec((NC, BN, d), lambda i: (0, i, 0))


def _full_spec(shape):
    return pl.BlockSpec(shape, lambda i: tuple(0 for _ in shape))


def _tc1(degp, xp, W1):
    dout = W1.shape[1]
    return pl.pallas_call(
        _tc1_body,
        grid=(GRID,),
        in_specs=[_part_spec(16), _row_spec(D_IN), _full_spec(W1.shape)],
        out_specs=[_row_spec(1), _row_spec(dout)],
        out_shape=[jax.ShapeDtypeStruct((NP, 1), jnp.float32),
                   jax.ShapeDtypeStruct((NP, dout), jnp.float32)],
    )(degp, xp, W1)


def _tc_mid(sp, g, dinv, b, W):
    din, dout = W.shape
    return pl.pallas_call(
        _tc_mid_body,
        grid=(GRID,),
        in_specs=[_part_spec(din), _row_spec(din), _row_spec(1),
                  _full_spec((1, din)), _full_spec(W.shape)],
        out_specs=_row_spec(dout),
        out_shape=jax.ShapeDtypeStruct((NP, dout), jnp.float32),
    )(sp, g, dinv, b, W)


def _tc_last(sp, g, dinv, b):
    d = g.shape[1]
    return pl.pallas_call(
        _tc_last_body,
        grid=(GRID,),
        in_specs=[_part_spec(d), _row_spec(d), _row_spec(1),
                  _full_spec((1, d))],
        out_specs=_row_spec(d),
        out_shape=jax.ShapeDtypeStruct((NP, d), jnp.float32),
    )(sp, g, dinv, b)


def kernel(x, edge_index, W1, b1, W2, b2, W3, b3):
    src = edge_index[0].astype(jnp.int32)
    dst = edge_index[1].astype(jnp.int32)
    pad = jnp.full((EPAD - E,), N, jnp.int32)
    src = jnp.concatenate([src, pad])
    dst = jnp.concatenate([dst, pad])
    xp = jnp.concatenate(
        [x, jnp.zeros((NP - N, x.shape[1]), jnp.float32)])

    degp = _deg_kernel(dst)
    dinv, g1 = _tc1(degp, xp, W1)
    s1 = _agg64(src, dst, g1)
    g2 = _tc_mid(s1, g1, dinv, b1.reshape(1, -1), W2)
    s2 = _agg32(src, dst, g2)
    g3 = _tc_mid(s2, g2, dinv, b2.reshape(1, -1), W3)
    s3 = _agg16(src, dst, g3)
    out = _tc_last(s3, g3, dinv, b3.reshape(1, -1))
    return out[:N]


# speedup vs baseline: 15.6358x; 15.6358x over previous
import functools

import jax
import jax.numpy as jnp
from jax import lax
from jax.experimental import pallas as pl
from jax.experimental.pallas import tpu as pltpu
from jax.experimental.pallas import tpu_sc as plsc

N = 10000
E = 320000
D_IN = 128

NC = 2
NS = 16
CHUNK = 128
NTILES = NC * NS

K = -(-E // (NTILES * CHUNK))
EPAD = NTILES * CHUNK * K
EPT = K * CHUNK

NP = 10240
ROWS_PT = NP // NS
NCOPY = ROWS_PT // CHUNK

BN = 1024
GRID = NP // BN

_MESH = dict(core_axis_name="c", subcore_axis_name="s")


def _make_agg(d):
    nlan = d // 16

    @functools.partial(
        pl.kernel,
        out_type=jax.ShapeDtypeStruct((NC, NP, d), jnp.float32),
        mesh=plsc.VectorSubcoreMesh(**_MESH),
        scratch_types=[
            pltpu.VMEM((CHUNK,), jnp.int32),
            pltpu.VMEM((CHUNK,), jnp.int32),
            pltpu.VMEM((CHUNK, d), jnp.float32),
            pltpu.VMEM_SHARED((NP, d), jnp.float32),
            pltpu.SemaphoreType.DMA,
        ],
        name=f"gcn_agg_d{d}",
        compiler_params=pltpu.CompilerParams(use_tc_tiling_on_sc=False),
    )
    def agg(src_hbm, dst_hbm, g_hbm, out_hbm, sidx, didx, rows, acc, sem):
        c = lax.axis_index("c")
        s = lax.axis_index("s")

        def zrow(i, _):
            rows[i // nlan, pl.ds((i % nlan) * 16, 16)] = jnp.zeros(
                (16,), jnp.float32)
            return 0
        lax.fori_loop(0, CHUNK * nlan, zrow, 0)
        row0 = s * ROWS_PT
        for j in range(NCOPY):
            pltpu.sync_copy(rows, acc.at[pl.ds(row0 + j * CHUNK, CHUNK)])
        plsc.subcore_barrier()

        base0 = (c * NS + s) * EPT

        def body(k, _):
            b = base0 + k * CHUNK
            pltpu.sync_copy(src_hbm.at[pl.ds(b, CHUNK)], sidx)
            pltpu.sync_copy(dst_hbm.at[pl.ds(b, CHUNK)], didx)
            pltpu.async_copy(g_hbm.at[sidx], rows, sem).wait()
            pltpu.sync_copy(rows, acc.at[didx], add=True)
            return 0
        lax.fori_loop(0, K, body, 0)

        plsc.subcore_barrier()
        pltpu.sync_copy(acc.at[pl.ds(row0, ROWS_PT)],
                        out_hbm.at[c, pl.ds(row0, ROWS_PT)])

    return agg


_agg64 = _make_agg(64)
_agg32 = _make_agg(32)
_agg16 = _make_agg(16)


@functools.partial(
    pl.kernel,
    out_type=jax.ShapeDtypeStruct((NC, NP, 16), jnp.float32),
    mesh=plsc.VectorSubcoreMesh(**_MESH),
    scratch_types=[
        pltpu.VMEM((CHUNK,), jnp.int32),
        pltpu.VMEM((CHUNK, 16), jnp.float32),
        pltpu.VMEM((CHUNK, 16), jnp.float32),
        pltpu.VMEM_SHARED((NP, 16), jnp.float32),
    ],
    name="gcn_degree",
    compiler_params=pltpu.CompilerParams(use_tc_tiling_on_sc=False),
)
def _deg_kernel(dst_hbm, out_hbm, didx, ones, zeros, acc):
    c = lax.axis_index("c")
    s = lax.axis_index("s")

    def fill(i, _):
        zeros[i, pl.ds(0, 16)] = jnp.zeros((16,), jnp.float32)
        ones[i, pl.ds(0, 16)] = jnp.ones((16,), jnp.float32)
        return 0
    lax.fori_loop(0, CHUNK, fill, 0)
    row0 = s * ROWS_PT
    for j in range(NCOPY):
        pltpu.sync_copy(zeros, acc.at[pl.ds(row0 + j * CHUNK, CHUNK)])
    plsc.subcore_barrier()

    base0 = (c * NS + s) * EPT

    def body(k, _):
        b = base0 + k * CHUNK
        pltpu.sync_copy(dst_hbm.at[pl.ds(b, CHUNK)], didx)
        pltpu.sync_copy(ones, acc.at[didx], add=True)
        return 0
    lax.fori_loop(0, K, body, 0)

    plsc.subcore_barrier()
    pltpu.sync_copy(acc.at[pl.ds(row0, ROWS_PT)],
                    out_hbm.at[c, pl.ds(row0, ROWS_PT)])


def _tc1_body(degp_ref, x_ref, w_ref, dinv_ref, g_ref):
    degp = degp_ref[...]
    deg = degp[0, :, 0:1] + degp[1, :, 0:1] + 1.0
    dinv = lax.rsqrt(deg)
    dinv_ref[...] = dinv
    g_ref[...] = dinv * jnp.dot(x_ref[...], w_ref[...],
                                preferred_element_type=jnp.float32)


def _tc_mid_body(sp_ref, g_ref, dinv_ref, b_ref, w_ref, gout_ref):
    dinv = dinv_ref[...]
    stot = sp_ref[0] + sp_ref[1] + g_ref[...]
    xn = jnp.maximum(dinv * stot + b_ref[...], 0.0)
    gout_ref[...] = dinv * jnp.dot(xn, w_ref[...],
                                   preferred_element_type=jnp.float32)


def _tc_last_body(sp_ref, g_ref, dinv_ref, b_ref, out_ref):
    out_ref[...] = dinv_ref[...] * (sp_ref[0] + sp_ref[1] + g_ref[...]) \
        + b_ref[...]


def _row_spec(d):
    return pl.BlockSpec((BN, d), lambda i: (i, 0))


def _part_spec(d):
    return pl.BlockSpec((NC, BN, d), lambda i: (0, i, 0))


def _full_spec(shape):
    return pl.BlockSpec(shape, lambda i: tuple(0 for _ in shape))


def _tc1(degp, xp, W1):
    dout = W1.shape[1]
    return pl.pallas_call(
        _tc1_body,
        grid=(GRID,),
        in_specs=[_part_spec(16), _row_spec(D_IN), _full_spec(W1.shape)],
        out_specs=[_row_spec(1), _row_spec(dout)],
        out_shape=[jax.ShapeDtypeStruct((NP, 1), jnp.float32),
                   jax.ShapeDtypeStruct((NP, dout), jnp.float32)],
    )(degp, xp, W1)


def _tc_mid(sp, g, dinv, b, W):
    din, dout = W.shape
    return pl.pallas_call(
        _tc_mid_body,
        grid=(GRID,),
        in_specs=[_part_spec(din), _row_spec(din), _row_spec(1),
                  _full_spec((1, din)), _full_spec(W.shape)],
        out_specs=_row_spec(dout),
        out_shape=jax.ShapeDtypeStruct((NP, dout), jnp.float32),
    )(sp, g, dinv, b, W)


def _tc_last(sp, g, dinv, b):
    d = g.shape[1]
    return pl.pallas_call(
        _tc_last_body,
        grid=(GRID,),
        in_specs=[_part_spec(d), _row_spec(d), _row_spec(1),
                  _full_spec((1, d))],
        out_specs=_row_spec(d),
        out_shape=jax.ShapeDtypeStruct((NP, d), jnp.float32),
    )(sp, g, dinv, b)


def kernel(x, edge_index, W1, b1, W2, b2, W3, b3):
    src = edge_index[0].astype(jnp.int32)
    dst = edge_index[1].astype(jnp.int32)
    pad = jnp.full((EPAD - E,), N, jnp.int32)
    src = jnp.concatenate([src, pad])
    dst = jnp.concatenate([dst, pad])
    xp = jnp.concatenate(
        [x, jnp.zeros((NP - N, x.shape[1]), jnp.float32)])

    degp = _deg_kernel(dst)
    dinv, g1 = _tc1(degp, xp, W1)
    s1 = _agg64(src, dst, g1)
    g2 = _tc_mid(s1, g1, dinv, b1.reshape(1, -1), W2)
    s2 = _agg32(src, dst, g2)
    g3 = _tc_mid(s2, g2, dinv, b2.reshape(1, -1), W3)
    s3 = _agg16(src, dst, g3)
    out = _tc_last(s3, g3, dinv, b3.reshape(1, -1))
    return out[:N]


# speedup vs baseline: 19.7481x; 1.2630x over previous
import functools

import jax
import jax.numpy as jnp
from jax import lax
from jax.experimental import pallas as pl
from jax.experimental.pallas import tpu as pltpu
from jax.experimental.pallas import tpu_sc as plsc

N = 10000
E = 320000
D_IN = 128

NC = 2
NS = 16
CHUNK = 128
NTILES = NC * NS

NBUF = 4
K = -(-E // (NTILES * CHUNK))
K = ((K + NBUF - 1) // NBUF) * NBUF
NGRP = K // NBUF
EPAD = NTILES * CHUNK * K
EPT = K * CHUNK

NP = 10240
ROWS_PT = NP // NS
NCOPY = ROWS_PT // CHUNK

BN = 1024
GRID = NP // BN

_MESH = dict(core_axis_name="c", subcore_axis_name="s")


def _make_agg(d):
    nlan = d // 16

    @functools.partial(
        pl.kernel,
        out_type=jax.ShapeDtypeStruct((NC, NP, d), jnp.float32),
        mesh=plsc.VectorSubcoreMesh(**_MESH),
        scratch_types=[
            pltpu.VMEM((K, CHUNK), jnp.int32),
            pltpu.VMEM((K, CHUNK), jnp.int32),
            pltpu.VMEM((NBUF, CHUNK, d), jnp.float32),
            pltpu.VMEM_SHARED((NP, d), jnp.float32),
        ] + [pltpu.SemaphoreType.DMA] * (2 * NBUF),
        name=f"gcn_agg_d{d}",
        compiler_params=pltpu.CompilerParams(use_tc_tiling_on_sc=False),
    )
    def agg(src_hbm, dst_hbm, g_hbm, out_hbm, sidx, didx, rows, acc,
            *sems):
        gsems, ssems = sems[:NBUF], sems[NBUF:]
        c = lax.axis_index("c")
        s = lax.axis_index("s")
        w = c * NS + s

        pltpu.sync_copy(src_hbm.at[w], sidx)
        pltpu.sync_copy(dst_hbm.at[w], didx)

        def zrow(i, _):
            rows[0, i // nlan, pl.ds((i % nlan) * 16, 16)] = jnp.zeros(
                (16,), jnp.float32)
            return 0
        lax.fori_loop(0, CHUNK * nlan, zrow, 0)
        row0 = s * ROWS_PT
        for j in range(NCOPY):
            pltpu.sync_copy(rows.at[0], acc.at[pl.ds(row0 + j * CHUNK,
                                                     CHUNK)])
        plsc.subcore_barrier()

        def gather(k, b):
            pltpu.async_copy(g_hbm.at[sidx.at[k]], rows.at[b], gsems[b])

        def gwait(b):
            pltpu.make_async_copy(g_hbm.at[sidx.at[0]], rows.at[b],
                                  gsems[b]).wait()

        def scatter(k, b):
            pltpu.async_copy(rows.at[b], acc.at[didx.at[k]], ssems[b],
                             add=True)

        def swait(b):
            pltpu.make_async_copy(rows.at[b], acc.at[didx.at[0]],
                                  ssems[b]).wait()

        for b in range(NBUF):
            gather(b, b)

        def grp(g, _):
            for b in range(NBUF):
                kprev = (g - 1) * NBUF + b
                gwait(b)
                scatter(kprev, b)
                swait(b)
                gather(g * NBUF + b, b)
            return 0
        lax.fori_loop(1, NGRP, grp, 0)

        for b in range(NBUF):
            kprev = (NGRP - 1) * NBUF + b
            gwait(b)
            scatter(kprev, b)
        for b in range(NBUF):
            swait(b)

        plsc.subcore_barrier()
        pltpu.sync_copy(acc.at[pl.ds(row0, ROWS_PT)],
                        out_hbm.at[c, pl.ds(row0, ROWS_PT)])

    return agg


_agg64 = _make_agg(64)
_agg32 = _make_agg(32)
_agg16 = _make_agg(16)


@functools.partial(
    pl.kernel,
    out_type=jax.ShapeDtypeStruct((NC, NP, 16), jnp.float32),
    mesh=plsc.VectorSubcoreMesh(**_MESH),
    scratch_types=[
        pltpu.VMEM((K, CHUNK), jnp.int32),
        pltpu.VMEM((CHUNK, 16), jnp.float32),
        pltpu.VMEM((CHUNK, 16), jnp.float32),
        pltpu.VMEM_SHARED((NP, 16), jnp.float32),
    ] + [pltpu.SemaphoreType.DMA] * NBUF,
    name="gcn_degree",
    compiler_params=pltpu.CompilerParams(use_tc_tiling_on_sc=False),
)
def _deg_kernel(dst_hbm, out_hbm, didx, ones, zeros, acc, *sems):
    c = lax.axis_index("c")
    s = lax.axis_index("s")
    w = c * NS + s

    pltpu.sync_copy(dst_hbm.at[w], didx)

    def fill(i, _):
        zeros[i, pl.ds(0, 16)] = jnp.zeros((16,), jnp.float32)
        ones[i, pl.ds(0, 16)] = jnp.ones((16,), jnp.float32)
        return 0
    lax.fori_loop(0, CHUNK, fill, 0)
    row0 = s * ROWS_PT
    for j in range(NCOPY):
        pltpu.sync_copy(zeros, acc.at[pl.ds(row0 + j * CHUNK, CHUNK)])
    plsc.subcore_barrier()

    def scat(k, b):
        pltpu.async_copy(ones, acc.at[didx.at[k]], sems[b], add=True)

    def swait(b):
        pltpu.make_async_copy(ones, acc.at[didx.at[0]], sems[b]).wait()

    for b in range(NBUF):
        scat(b, b)

    def grp(g, _):
        for b in range(NBUF):
            swait(b)
            scat(g * NBUF + b, b)
        return 0
    lax.fori_loop(1, NGRP, grp, 0)

    for b in range(NBUF):
        swait(b)

    plsc.subcore_barrier()
    pltpu.sync_copy(acc.at[pl.ds(row0, ROWS_PT)],
                    out_hbm.at[c, pl.ds(row0, ROWS_PT)])


def _tc1_body(degp_ref, x_ref, w_ref, dinv_ref, g_ref):
    degp = degp_ref[...]
    deg = degp[0, :, 0:1] + degp[1, :, 0:1] + 1.0
    dinv = lax.rsqrt(deg)
    dinv_ref[...] = dinv
    g_ref[...] = dinv * jnp.dot(x_ref[...], w_ref[...],
                                preferred_element_type=jnp.float32)


def _tc_mid_body(sp_ref, g_ref, dinv_ref, b_ref, w_ref, gout_ref):
    dinv = dinv_ref[...]
    stot = sp_ref[0] + sp_ref[1] + g_ref[...]
    xn = jnp.maximum(dinv * stot + b_ref[...], 0.0)
    gout_ref[...] = dinv * jnp.dot(xn, w_ref[...],
                                   preferred_element_type=jnp.float32)


def _tc_last_body(sp_ref, g_ref, dinv_ref, b_ref, out_ref):
    out_ref[...] = dinv_ref[...] * (sp_ref[0] + sp_ref[1] + g_ref[...]) \
        + b_ref[...]


def _row_spec(d):
    return pl.BlockSpec((BN, d), lambda i: (i, 0))


def _part_spec(d):
    return pl.BlockSpec((NC, BN, d), lambda i: (0, i, 0))


def _full_spec(shape):
    return pl.BlockSpec(shape, lambda i: tuple(0 for _ in shape))


def _tc1(degp, xp, W1):
    dout = W1.shape[1]
    return pl.pallas_call(
        _tc1_body,
        grid=(GRID,),
        in_specs=[_part_spec(16), _row_spec(D_IN), _full_spec(W1.shape)],
        out_specs=[_row_spec(1), _row_spec(dout)],
        out_shape=[jax.ShapeDtypeStruct((NP, 1), jnp.float32),
                   jax.ShapeDtypeStruct((NP, dout), jnp.float32)],
    )(degp, xp, W1)


def _tc_mid(sp, g, dinv, b, W):
    din, dout = W.shape
    return pl.pallas_call(
        _tc_mid_body,
        grid=(GRID,),
        in_specs=[_part_spec(din), _row_spec(din), _row_spec(1),
                  _full_spec((1, din)), _full_spec(W.shape)],
        out_specs=_row_spec(dout),
        out_shape=jax.ShapeDtypeStruct((NP, dout), jnp.float32),
    )(sp, g, dinv, b, W)


def _tc_last(sp, g, dinv, b):
    d = g.shape[1]
    return pl.pallas_call(
        _tc_last_body,
        grid=(GRID,),
        in_specs=[_part_spec(d), _row_spec(d), _row_spec(1),
                  _full_spec((1, d))],
        out_specs=_row_spec(d),
        out_shape=jax.ShapeDtypeStruct((NP, d), jnp.float32),
    )(sp, g, dinv, b)


def kernel(x, edge_index, W1, b1, W2, b2, W3, b3):
    src = edge_index[0].astype(jnp.int32)
    dst = edge_index[1].astype(jnp.int32)
    pad = jnp.full((EPAD - E,), N, jnp.int32)
    src = jnp.concatenate([src, pad]).reshape(NTILES, K, CHUNK)
    dst = jnp.concatenate([dst, pad]).reshape(NTILES, K, CHUNK)
    xp = jnp.concatenate(
        [x, jnp.zeros((NP - N, x.shape[1]), jnp.float32)])

    degp = _deg_kernel(dst)
    dinv, g1 = _tc1(degp, xp, W1)
    s1 = _agg64(src, dst, g1)
    g2 = _tc_mid(s1, g1, dinv, b1.reshape(1, -1), W2)
    s2 = _agg32(src, dst, g2)
    g3 = _tc_mid(s2, g2, dinv, b2.reshape(1, -1), W3)
    s3 = _agg16(src, dst, g3)
    out = _tc_last(s3, g3, dinv, b3.reshape(1, -1))
    return out[:N]


# speedup vs baseline: 38.9830x; 1.9740x over previous
import functools

import jax
import jax.numpy as jnp
from jax import lax
from jax.experimental import pallas as pl
from jax.experimental.pallas import tpu as pltpu
from jax.experimental.pallas import tpu_sc as plsc

N = 10000
E = 320000
D_IN = 128

NC = 2
NS = 16
CHUNK = 128
NTILES = NC * NS

NBUF = 4
K = -(-E // (NTILES * CHUNK))
K = ((K + NBUF - 1) // NBUF) * NBUF
NGRP = K // NBUF
EPAD = NTILES * CHUNK * K
EPT = K * CHUNK

NP = 10240
ROWS_PT = NP // NS
NCOPY = ROWS_PT // CHUNK

BN = 1024
GRID = NP // BN

_MESH = dict(core_axis_name="c", subcore_axis_name="s")


def _make_agg(d, npass):
    dcol = d // npass
    nlan = dcol // 16

    @functools.partial(
        pl.kernel,
        out_type=jax.ShapeDtypeStruct((NC, NP, d), jnp.float32),
        mesh=plsc.VectorSubcoreMesh(**_MESH),
        scratch_types=[
            pltpu.VMEM((K, CHUNK), jnp.int32),
            pltpu.VMEM((K, CHUNK), jnp.int32),
            pltpu.VMEM((NBUF, CHUNK, dcol), jnp.float32),
            pltpu.VMEM_SHARED((NP, dcol), jnp.float32),
            pltpu.VMEM_SHARED((NP, dcol), jnp.float32),
        ] + [pltpu.SemaphoreType.DMA] * (2 * NBUF),
        name=f"gcn_agg_d{d}",
        compiler_params=pltpu.CompilerParams(use_tc_tiling_on_sc=False),
    )
    def agg(src_hbm, dst_hbm, g_hbm, out_hbm, sidx, didx, rows, acc,
            gtab, *sems):
        gsems, ssems = sems[:NBUF], sems[NBUF:]
        c = lax.axis_index("c")
        s = lax.axis_index("s")
        w = c * NS + s
        row0 = s * ROWS_PT

        pltpu.sync_copy(src_hbm.at[w], sidx)
        pltpu.sync_copy(dst_hbm.at[w], didx)

        def gather(k, b):
            pltpu.async_copy(gtab.at[sidx.at[k]], rows.at[b], gsems[b])

        def gwait(b):
            pltpu.make_async_copy(gtab.at[sidx.at[0]], rows.at[b],
                                  gsems[b]).wait()

        def scatter(k, b):
            pltpu.async_copy(rows.at[b], acc.at[didx.at[k]], ssems[b],
                             add=True)

        def swait(b):
            pltpu.make_async_copy(rows.at[b], acc.at[didx.at[0]],
                                  ssems[b]).wait()

        for p in range(npass):
            pltpu.sync_copy(
                g_hbm.at[pl.ds(row0, ROWS_PT), pl.ds(p * dcol, dcol)],
                gtab.at[pl.ds(row0, ROWS_PT)])

            def zrow(i, _):
                rows[0, i // nlan, pl.ds((i % nlan) * 16, 16)] = \
                    jnp.zeros((16,), jnp.float32)
                return 0
            lax.fori_loop(0, CHUNK * nlan, zrow, 0)
            for j in range(NCOPY):
                pltpu.sync_copy(rows.at[0],
                                acc.at[pl.ds(row0 + j * CHUNK, CHUNK)])
            plsc.subcore_barrier()

            for b in range(NBUF):
                gather(b, b)

            def grp(g, _):
                for b in range(NBUF):
                    kprev = (g - 1) * NBUF + b
                    gwait(b)
                    scatter(kprev, b)
                    swait(b)
                    gather(g * NBUF + b, b)
                return 0
            lax.fori_loop(1, NGRP, grp, 0)

            for b in range(NBUF):
                kprev = (NGRP - 1) * NBUF + b
                gwait(b)
                scatter(kprev, b)
            for b in range(NBUF):
                swait(b)

            plsc.subcore_barrier()
            pltpu.sync_copy(
                acc.at[pl.ds(row0, ROWS_PT)],
                out_hbm.at[c, pl.ds(row0, ROWS_PT), pl.ds(p * dcol, dcol)])

    return agg


_agg64 = _make_agg(64, 2)
_agg32 = _make_agg(32, 1)
_agg16 = _make_agg(16, 1)


@functools.partial(
    pl.kernel,
    out_type=jax.ShapeDtypeStruct((NC, NP, 16), jnp.float32),
    mesh=plsc.VectorSubcoreMesh(**_MESH),
    scratch_types=[
        pltpu.VMEM((K, CHUNK), jnp.int32),
        pltpu.VMEM((CHUNK, 16), jnp.float32),
        pltpu.VMEM((CHUNK, 16), jnp.float32),
        pltpu.VMEM_SHARED((NP, 16), jnp.float32),
    ] + [pltpu.SemaphoreType.DMA] * NBUF,
    name="gcn_degree",
    compiler_params=pltpu.CompilerParams(use_tc_tiling_on_sc=False),
)
def _deg_kernel(dst_hbm, out_hbm, didx, ones, zeros, acc, *sems):
    c = lax.axis_index("c")
    s = lax.axis_index("s")
    w = c * NS + s

    pltpu.sync_copy(dst_hbm.at[w], didx)

    def fill(i, _):
        zeros[i, pl.ds(0, 16)] = jnp.zeros((16,), jnp.float32)
        ones[i, pl.ds(0, 16)] = jnp.ones((16,), jnp.float32)
        return 0
    lax.fori_loop(0, CHUNK, fill, 0)
    row0 = s * ROWS_PT
    for j in range(NCOPY):
        pltpu.sync_copy(zeros, acc.at[pl.ds(row0 + j * CHUNK, CHUNK)])
    plsc.subcore_barrier()

    def scat(k, b):
        pltpu.async_copy(ones, acc.at[didx.at[k]], sems[b], add=True)

    def swait(b):
        pltpu.make_async_copy(ones, acc.at[didx.at[0]], sems[b]).wait()

    for b in range(NBUF):
        scat(b, b)

    def grp(g, _):
        for b in range(NBUF):
            swait(b)
            scat(g * NBUF + b, b)
        return 0
    lax.fori_loop(1, NGRP, grp, 0)

    for b in range(NBUF):
        swait(b)

    plsc.subcore_barrier()
    pltpu.sync_copy(acc.at[pl.ds(row0, ROWS_PT)],
                    out_hbm.at[c, pl.ds(row0, ROWS_PT)])


def _tc1_body(degp_ref, x_ref, w_ref, dinv_ref, g_ref):
    degp = degp_ref[...]
    deg = degp[0, :, 0:1] + degp[1, :, 0:1] + 1.0
    dinv = lax.rsqrt(deg)
    dinv_ref[...] = dinv
    g_ref[...] = dinv * jnp.dot(x_ref[...], w_ref[...],
                                preferred_element_type=jnp.float32)


def _tc_mid_body(sp_ref, g_ref, dinv_ref, b_ref, w_ref, gout_ref):
    dinv = dinv_ref[...]
    stot = sp_ref[0] + sp_ref[1] + g_ref[...]
    xn = jnp.maximum(dinv * stot + b_ref[...], 0.0)
    gout_ref[...] = dinv * jnp.dot(xn, w_ref[...],
                                   preferred_element_type=jnp.float32)


def _tc_last_body(sp_ref, g_ref, dinv_ref, b_ref, out_ref):
    out_ref[...] = dinv_ref[...] * (sp_ref[0] + sp_ref[1] + g_ref[...]) \
        + b_ref[...]


def _row_spec(d):
    return pl.BlockSpec((BN, d), lambda i: (i, 0))


def _part_spec(d):
    return pl.BlockSpec((NC, BN, d), lambda i: (0, i, 0))


def _full_spec(shape):
    return pl.BlockSpec(shape, lambda i: tuple(0 for _ in shape))


def _tc1(degp, xp, W1):
    dout = W1.shape[1]
    return pl.pallas_call(
        _tc1_body,
        grid=(GRID,),
        in_specs=[_part_spec(16), _row_spec(D_IN), _full_spec(W1.shape)],
        out_specs=[_row_spec(1), _row_spec(dout)],
        out_shape=[jax.ShapeDtypeStruct((NP, 1), jnp.float32),
                   jax.ShapeDtypeStruct((NP, dout), jnp.float32)],
    )(degp, xp, W1)


def _tc_mid(sp, g, dinv, b, W):
    din, dout = W.shape
    return pl.pallas_call(
        _tc_mid_body,
        grid=(GRID,),
        in_specs=[_part_spec(din), _row_spec(din), _row_spec(1),
                  _full_spec((1, din)), _full_spec(W.shape)],
        out_specs=_row_spec(dout),
        out_shape=jax.ShapeDtypeStruct((NP, dout), jnp.float32),
    )(sp, g, dinv, b, W)


def _tc_last(sp, g, dinv, b):
    d = g.shape[1]
    return pl.pallas_call(
        _tc_last_body,
        grid=(GRID,),
        in_specs=[_part_spec(d), _row_spec(d), _row_spec(1),
                  _full_spec((1, d))],
        out_specs=_row_spec(d),
        out_shape=jax.ShapeDtypeStruct((NP, d), jnp.float32),
    )(sp, g, dinv, b)


def kernel(x, edge_index, W1, b1, W2, b2, W3, b3):
    src = edge_index[0].astype(jnp.int32)
    dst = edge_index[1].astype(jnp.int32)
    pad = jnp.full((EPAD - E,), N, jnp.int32)
    src = jnp.concatenate([src, pad]).reshape(NTILES, K, CHUNK)
    dst = jnp.concatenate([dst, pad]).reshape(NTILES, K, CHUNK)
    xp = jnp.concatenate(
        [x, jnp.zeros((NP - N, x.shape[1]), jnp.float32)])

    degp = _deg_kernel(dst)
    dinv, g1 = _tc1(degp, xp, W1)
    s1 = _agg64(src, dst, g1)
    g2 = _tc_mid(s1, g1, dinv, b1.reshape(1, -1), W2)
    s2 = _agg32(src, dst, g2)
    g3 = _tc_mid(s2, g2, dinv, b2.reshape(1, -1), W3)
    s3 = _agg16(src, dst, g3)
    out = _tc_last(s3, g3, dinv, b3.reshape(1, -1))
    return out[:N]
